# baseline (speedup 1.0000x reference)
import functools

import jax
import jax.numpy as jnp
from jax import lax
from jax.experimental import pallas as pl
from jax.experimental.pallas import tpu as pltpu
from jax.experimental.pallas import tpu_sc as plsc

CORPUS = 1000000
B = 16384
D = 32
PACK = 128 // D
M4 = CORPUS // PACK
NC = 2
NS = 16
NW = NC * NS
BPW = B // NW
SB = 128
NSB = BPW // SB
LANES = 16
OROWS = BPW * D // 128


def _extract(idx_ref, idx_base, buf, out_ref, out_base):
    lane = jnp.arange(LANES, dtype=jnp.int32)

    def body(g, carry):
        k16 = lane + g * LANES
        idx16 = idx_ref[pl.ds(idx_base + g * LANES, LANES)]
        off16 = (idx16 & (PACK - 1)) << 5
        flat_base = (k16 + out_base) * D
        for c in range(D):
            v = plsc.load_gather(buf, [k16, off16 + c])
            flat = flat_base + c
            plsc.store_scatter(out_ref, [flat >> 7, flat & 127], v)
        return carry

    lax.fori_loop(0, SB // LANES, body, 0)


def _glove_lookup(i, j, W4, b):
    mesh = plsc.VectorSubcoreMesh(core_axis_name="c", subcore_axis_name="s")

    @functools.partial(
        pl.kernel,
        mesh=mesh,
        compiler_params=pltpu.CompilerParams(needs_layout_passes=False),
        out_type=(
            jax.ShapeDtypeStruct((B * D // 128, 128), jnp.float32),
            jax.ShapeDtypeStruct((B * D // 128, 128), jnp.float32),
            jax.ShapeDtypeStruct((B,), jnp.float32),
            jax.ShapeDtypeStruct((B,), jnp.float32),
        ),
        scratch_types=[
            pltpu.VMEM((BPW,), jnp.int32),
            pltpu.VMEM((BPW,), jnp.int32),
            pltpu.VMEM((BPW,), jnp.int32),
            pltpu.VMEM((BPW,), jnp.int32),
            pltpu.VMEM((SB, PACK * D), jnp.float32),
            pltpu.VMEM((SB, PACK * D), jnp.float32),
            pltpu.VMEM((OROWS, 128), jnp.float32),
            pltpu.VMEM((OROWS, 128), jnp.float32),
            pltpu.VMEM((BPW,), jnp.float32),
            pltpu.VMEM((BPW,), jnp.float32),
            pltpu.SemaphoreType.DMA,
            pltpu.SemaphoreType.DMA,
            pltpu.SemaphoreType.DMA,
            pltpu.SemaphoreType.DMA,
            pltpu.SemaphoreType.DMA,
        ],
    )
    def k(i_hbm, j_hbm, w4_hbm, b_hbm,
          wi_hbm, wj_hbm, bi_hbm, bj_hbm,
          idx_i, idx_j, q_i, q_j, buf_i, buf_j, rows_i, rows_j, bv_i, bv_j,
          sem_idx, sem_i, sem_j, sem_b, sem_out):
        wid = lax.axis_index("s") * NC + lax.axis_index("c")
        base = wid * BPW
        ld_i = pltpu.async_copy(i_hbm.at[pl.ds(base, BPW)], idx_i, sem_idx)
        ld_j = pltpu.async_copy(j_hbm.at[pl.ds(base, BPW)], idx_j, sem_idx)
        ld_i.wait()
        ld_j.wait()

        def qbody(g, carry):
            sl = pl.ds(g * LANES, LANES)
            q_i[sl] = idx_i[sl] >> 2
            q_j[sl] = idx_j[sl] >> 2
            return carry

        lax.fori_loop(0, BPW // LANES, qbody, 0)

        def fire_w(q_ref, sb, buf, sem):
            sl = pl.ds(sb * SB, SB)
            return pltpu.async_copy(w4_hbm.at[q_ref.at[sl]], buf, sem)

        bcps = []
        for c in range(NSB):
            sl = pl.ds(c * SB, SB)
            bcps.append(pltpu.async_copy(b_hbm.at[idx_i.at[sl]], bv_i.at[sl], sem_b))
            bcps.append(pltpu.async_copy(b_hbm.at[idx_j.at[sl]], bv_j.at[sl], sem_b))

        gi = fire_w(q_i, 0, buf_i, sem_i)
        gj = fire_w(q_j, 0, buf_j, sem_j)
        for sb in range(NSB):
            gi.wait()
            _extract(idx_i, sb * SB, buf_i, rows_i, sb * SB)
            if sb + 1 < NSB:
                gi = fire_w(q_i, sb + 1, buf_i, sem_i)
            gj.wait()
            _extract(idx_j, sb * SB, buf_j, rows_j, sb * SB)
            if sb + 1 < NSB:
                gj = fire_w(q_j, sb + 1, buf_j, sem_j)

        obase = wid * OROWS
        st_i = pltpu.async_copy(rows_i, wi_hbm.at[pl.ds(obase, OROWS)], sem_out)
        st_j = pltpu.async_copy(rows_j, wj_hbm.at[pl.ds(obase, OROWS)], sem_out)

        for cp in bcps:
            cp.wait()
        st_bi = pltpu.async_copy(bv_i, bi_hbm.at[pl.ds(base, BPW)], sem_out)
        st_bj = pltpu.async_copy(bv_j, bj_hbm.at[pl.ds(base, BPW)], sem_out)

        st_i.wait()
        st_j.wait()
        st_bi.wait()
        st_bj.wait()

    return k(i, j, W4, b)


def kernel(i, j, W, b):
    W4 = jnp.reshape(W, (M4, PACK * D))
    wi4, wj4, b_i, b_j = _glove_lookup(i, j, W4, b)
    return (jnp.reshape(wi4, (B, D)), jnp.reshape(wj4, (B, D)), b_i, b_j)


# speedup vs baseline: 1.0753x; 1.0753x over previous
import functools

import jax
import jax.numpy as jnp
from jax import lax
from jax.experimental import pallas as pl
from jax.experimental.pallas import tpu as pltpu
from jax.experimental.pallas import tpu_sc as plsc

B = 16384
D = 32
NC = 2
NS = 16
NW = NC * NS
BPW = B // NW
CH = 128
NCH = BPW // CH


def _glove_lookup(i, j, W, b):
    mesh = plsc.VectorSubcoreMesh(core_axis_name="c", subcore_axis_name="s")

    @functools.partial(
        pl.kernel,
        mesh=mesh,
        compiler_params=pltpu.CompilerParams(use_tc_tiling_on_sc=False),
        out_type=(
            jax.ShapeDtypeStruct((B, D), jnp.float32),
            jax.ShapeDtypeStruct((B, D), jnp.float32),
            jax.ShapeDtypeStruct((B,), jnp.float32),
            jax.ShapeDtypeStruct((B,), jnp.float32),
        ),
        scratch_types=[
            pltpu.VMEM((BPW,), jnp.int32),
            pltpu.VMEM((BPW,), jnp.int32),
            pltpu.VMEM((BPW, D), jnp.float32),
            pltpu.VMEM((BPW, D), jnp.float32),
            pltpu.VMEM((BPW,), jnp.float32),
            pltpu.VMEM((BPW,), jnp.float32),
            pltpu.SemaphoreType.DMA,
        ],
    )
    def k(i_hbm, j_hbm, w_hbm, b_hbm,
          wi_hbm, wj_hbm, bi_hbm, bj_hbm,
          idx_i, idx_j, rows_i, rows_j, bv_i, bv_j, sem):
        wid = lax.axis_index("s") * NC + lax.axis_index("c")
        base = wid * BPW
        pltpu.sync_copy(i_hbm.at[pl.ds(base, BPW)], idx_i)
        pltpu.sync_copy(j_hbm.at[pl.ds(base, BPW)], idx_j)
        copies = []
        for c in range(NCH):
            sl = pl.ds(c * CH, CH)
            copies.append(pltpu.async_copy(w_hbm.at[idx_i.at[sl]], rows_i.at[sl], sem))
            copies.append(pltpu.async_copy(w_hbm.at[idx_j.at[sl]], rows_j.at[sl], sem))
            copies.append(pltpu.async_copy(b_hbm.at[idx_i.at[sl]], bv_i.at[sl], sem))
            copies.append(pltpu.async_copy(b_hbm.at[idx_j.at[sl]], bv_j.at[sl], sem))
        for cp in copies:
            cp.wait()
        out_sl = pl.ds(base, BPW)
        pltpu.sync_copy(rows_i, wi_hbm.at[out_sl])
        pltpu.sync_copy(rows_j, wj_hbm.at[out_sl])
        pltpu.sync_copy(bv_i, bi_hbm.at[out_sl])
        pltpu.sync_copy(bv_j, bj_hbm.at[out_sl])

    return k(i, j, W, b)


def kernel(i, j, W, b):
    return _glove_lookup(i, j, W, b)
